# baseline (speedup 1.0000x reference)
import functools

import jax
import jax.numpy as jnp
from jax import lax
from jax.experimental import pallas as pl
from jax.experimental.pallas import tpu as pltpu
from jax.experimental.pallas import tpu_sc as plsc

N = 10000
E = 320000
H = 64
R = 8
KN = N * R

NC = 2
NS = 16
NW = NC * NS
L = 16

SB = 80
C = 400
SUBS = C // SB
EPW = E // NW
NCHUNK = EPW // C
CP = 2000
ZROWS = 125
RPS = N // NS


def _sc_mesh():
    return plsc.VectorSubcoreMesh(
        core_axis_name="c", subcore_axis_name="s",
        num_cores=NC, num_subcores=NS)


_SC_PARAMS = pltpu.CompilerParams(
    needs_layout_passes=False, use_tc_tiling_on_sc=False)


@functools.partial(
    pl.kernel,
    out_type=(
        jax.ShapeDtypeStruct((E,), jnp.int32),
        jax.ShapeDtypeStruct((E,), jnp.int32),
        jax.ShapeDtypeStruct((NW, KN), jnp.float32),
    ),
    mesh=_sc_mesh(),
    scratch_types=[
        pltpu.VMEM((KN,), jnp.float32),
        pltpu.VMEM((CP,), jnp.int32),
        pltpu.VMEM((CP,), jnp.int32),
        pltpu.VMEM((CP,), jnp.int32),
        pltpu.VMEM((CP,), jnp.int32),
        pltpu.VMEM((CP,), jnp.int32),
    ],
    compiler_params=_SC_PARAMS,
)
def _sc_prep(src, dst, et, zflat, srckey, dstkey, hist,
             hist_v, src_v, dst_v, typ_v, sk_v, dk_v):
    cc = lax.axis_index("c")
    ss = lax.axis_index("s")
    wid = ss * NC + cc
    base = wid * EPW
    for i in range(KN // 8000):
        pltpu.sync_copy(zflat, hist_v.at[pl.ds(i * 8000, 8000)])
    ones = jnp.ones((L,), jnp.float32)

    def chunk(i, carry):
        off = base + i * CP
        pltpu.sync_copy(src.at[pl.ds(off, CP)], src_v)
        pltpu.sync_copy(dst.at[pl.ds(off, CP)], dst_v)
        pltpu.sync_copy(et.at[pl.ds(off, CP)], typ_v)
        for k in range(CP // L):
            sl = pl.ds(k * L, L)
            t = typ_v[sl]
            sk_v[sl] = src_v[sl] * R + t
            dk = dst_v[sl] * R + t
            dk_v[sl] = dk
            plsc.addupdate_scatter(hist_v, [dk], ones)
        pltpu.sync_copy(sk_v, srckey.at[pl.ds(off, CP)])
        pltpu.sync_copy(dk_v, dstkey.at[pl.ds(off, CP)])
        return carry

    lax.fori_loop(0, EPW // CP, chunk, 0)
    pltpu.sync_copy(hist_v, hist.at[wid])


KSL = KN // NS


@functools.partial(
    pl.kernel,
    out_type=jax.ShapeDtypeStruct((E,), jnp.float32),
    mesh=_sc_mesh(),
    scratch_types=[
        pltpu.VMEM((KN,), jnp.float32),
        pltpu.VMEM((KSL,), jnp.float32),
        pltpu.VMEM((KSL,), jnp.float32),
        pltpu.VMEM((CP,), jnp.int32),
        pltpu.VMEM((CP,), jnp.float32),
        pltpu.VMEM_SHARED((KN,), jnp.float32),
    ],
    compiler_params=_SC_PARAMS,
)
def _sc_scale(dstkey, hist, scale, recip_v, acc_v, tmp_v, dk_v, s_v,
              recip_sh):
    cc = lax.axis_index("c")
    ss = lax.axis_index("s")
    wid = ss * NC + cc
    base = wid * EPW
    koff = ss * KSL
    pltpu.sync_copy(hist.at[0, pl.ds(koff, KSL)], acc_v)

    def red(w, carry):
        pltpu.sync_copy(hist.at[w, pl.ds(koff, KSL)], tmp_v)
        for k in range(KSL // L):
            sl = pl.ds(k * L, L)
            acc_v[sl] = acc_v[sl] + tmp_v[sl]
        return carry

    lax.fori_loop(1, NW, red, 0)
    one = jnp.ones((L,), jnp.float32)
    for k in range(KSL // L):
        sl = pl.ds(k * L, L)
        acc_v[sl] = one / jnp.maximum(acc_v[sl], one)
    pltpu.sync_copy(acc_v, recip_sh.at[pl.ds(koff, KSL)])
    plsc.subcore_barrier()
    pltpu.sync_copy(recip_sh, recip_v)

    def chunk(i, carry):
        off = base + i * CP
        pltpu.sync_copy(dstkey.at[pl.ds(off, CP)], dk_v)
        for k in range(CP // L):
            sl = pl.ds(k * L, L)
            s_v[sl] = plsc.load_gather(recip_v, [dk_v[sl]])
        pltpu.sync_copy(s_v, scale.at[pl.ds(off, CP)])
        return carry

    lax.fori_loop(0, EPW // CP, chunk, 0)


@functools.partial(
    pl.kernel,
    out_type=jax.ShapeDtypeStruct((NC, N, H), jnp.float32),
    mesh=_sc_mesh(),
    scratch_types=[
        pltpu.VMEM((C, H), jnp.float32),
        pltpu.VMEM((C, H), jnp.float32),
        pltpu.VMEM((C, H), jnp.float32),
        pltpu.VMEM((C,), jnp.int32),
        pltpu.VMEM((C,), jnp.int32),
        pltpu.VMEM((C,), jnp.int32),
        pltpu.VMEM((SUBS, SB), jnp.int32),
        pltpu.VMEM((SUBS, SB), jnp.int32),
        pltpu.VMEM((SUBS, SB), jnp.int32),
        pltpu.VMEM((C,), jnp.float32),
        pltpu.VMEM((C,), jnp.float32),
        pltpu.VMEM((C,), jnp.float32),
        pltpu.VMEM_SHARED((N, H), jnp.float32),
        pltpu.SemaphoreType.DMA,
        pltpu.SemaphoreType.DMA,
        pltpu.SemaphoreType.DMA,
        pltpu.SemaphoreType.DMA,
        pltpu.SemaphoreType.DMA,
        pltpu.SemaphoreType.DMA,
    ],
    compiler_params=_SC_PARAMS,
)
def _sc_edge(y, srckey, dst2, scale, zrows, out,
             rows_a, rows_b, rows_z, sk_a, sk_b, sk_z, dk_a, dk_b, dk_z,
             s_a, s_b, s_z, agg_sh, gsem_a, gsem_b, gsem_z,
             ssem_a, ssem_b, ssem_z):
    cc = lax.axis_index("c")
    ss = lax.axis_index("s")
    wid = ss * NC + cc
    base = wid * EPW
    for j in range(RPS // ZROWS):
        pltpu.sync_copy(zrows, agg_sh.at[pl.ds(ss * RPS + j * ZROWS, ZROWS)])
    plsc.subcore_barrier()

    def loadmeta(n, sk_v, dk_v, s_v):
        off = base + n * C
        row = (base + n * C) // SB
        pltpu.sync_copy(srckey.at[pl.ds(off, C)], sk_v)
        pltpu.sync_copy(dst2.at[pl.ds(row, SUBS)], dk_v)
        pltpu.sync_copy(scale.at[pl.ds(off, C)], s_v)

    def gather_descs(sk_v, rows_v, sem):
        return [
            pltpu.make_async_copy(
                y.at[sk_v.at[pl.ds(j * SB, SB)]],
                rows_v.at[pl.ds(j * SB, SB)], sem)
            for j in range(SUBS)
        ]

    def scatter_descs(rows_v, dk_v, sem):
        return [
            pltpu.make_async_copy(
                rows_v.at[pl.ds(j * SB, SB)],
                agg_sh.at[dk_v.at[j]], sem)
            for j in range(SUBS)
        ]

    def issue(descs, add=False):
        for d in descs:
            d.start(add=add)

    def drain(descs):
        for d in descs:
            d.wait()

    def mul_rows(rows_v, s_v):
        def body(e):
            eidx = jnp.full((L,), e, jnp.int32)
            sv = plsc.load_gather(s_v, [eidx])
            for j in range(H // L):
                sl = pl.ds(j * L, L)
                rows_v[e, sl] = rows_v[e, sl] * sv
        plsc.parallel_loop(0, C, 1, unroll=16)(body)

    BUFS = ((rows_a, sk_a, dk_a, s_a, gsem_a, ssem_a),
            (rows_b, sk_b, dk_b, s_b, gsem_b, ssem_b),
            (rows_z, sk_z, dk_z, s_z, gsem_z, ssem_z))

    def refill(n, buf):
        rows_v, sk_v, dk_v, s_v, gsem, _ = buf
        loadmeta(n, sk_v, dk_v, s_v)
        issue(gather_descs(sk_v, rows_v, gsem))

    def process(buf):
        rows_v, sk_v, dk_v, s_v, gsem, ssem = buf
        drain(gather_descs(sk_v, rows_v, gsem))
        mul_rows(rows_v, s_v)
        issue(scatter_descs(rows_v, dk_v, ssem), add=True)

    def wait_scat(buf):
        rows_v, _, dk_v, _, _, ssem = buf
        drain(scatter_descs(rows_v, dk_v, ssem))

    refill(0, BUFS[0])
    refill(1, BUFS[1])

    def trip(i, carry):
        a, b, z = BUFS
        process(a)

        @pl.when(i > 0)
        def _():
            wait_scat(z)
        refill(3 * i + 2, z)

        process(b)

        @pl.when(3 * i + 3 < NCHUNK)
        def _():
            wait_scat(a)
            refill(3 * i + 3, a)

        process(z)

        @pl.when(3 * i + 4 < NCHUNK)
        def _():
            wait_scat(b)
            refill(3 * i + 4, b)
        return carry

    lax.fori_loop(0, NCHUNK // 3, trip, 0)
    process(BUFS[0])
    wait_scat(BUFS[0])
    wait_scat(BUFS[1])
    wait_scat(BUFS[2])
    plsc.subcore_barrier()

    @pl.when(ss == 0)
    def _():
        pltpu.sync_copy(agg_sh, out.at[cc])


BM = 1000


def _tc_lin1(x, w, root, b):
    def body(x_ref, w_ref, root_ref, b_ref, rt_ref, y_ref):
        feat = jnp.concatenate([x_ref[0], x_ref[1]], axis=1)
        rt_ref[...] = jnp.dot(
            feat, root_ref[...], preferred_element_type=jnp.float32
        ) + b_ref[...]
        for r in range(R):
            y_ref[:, r, :] = jnp.dot(
                feat, w_ref[r], preferred_element_type=jnp.float32)

    return pl.pallas_call(
        body,
        grid=(N // BM,),
        in_specs=[
            pl.BlockSpec((2, BM, H), lambda i: (0, i, 0)),
            pl.BlockSpec((R, 2 * H, H), lambda i: (0, 0, 0)),
            pl.BlockSpec((2 * H, H), lambda i: (0, 0)),
            pl.BlockSpec((1, H), lambda i: (0, 0)),
        ],
        out_specs=[
            pl.BlockSpec((BM, H), lambda i: (i, 0)),
            pl.BlockSpec((BM, R, H), lambda i: (i, 0, 0)),
        ],
        out_shape=[
            jax.ShapeDtypeStruct((N, H), jnp.float32),
            jax.ShapeDtypeStruct((N, R, H), jnp.float32),
        ],
    )(x, w, root, b)


def _tc_comb(rt, part, w, root, b):
    def body(rt_ref, p_ref, w_ref, root_ref, b_ref, rt2_ref, y2_ref):
        h = jnp.maximum(rt_ref[...] + p_ref[0] + p_ref[1], 0.0)
        rt2_ref[...] = jnp.dot(
            h, root_ref[...], preferred_element_type=jnp.float32
        ) + b_ref[...]
        for r in range(R):
            y2_ref[:, r, :] = jnp.dot(
                h, w_ref[r], preferred_element_type=jnp.float32)

    return pl.pallas_call(
        body,
        grid=(N // BM,),
        in_specs=[
            pl.BlockSpec((BM, H), lambda i: (i, 0)),
            pl.BlockSpec((NC, BM, H), lambda i: (0, i, 0)),
            pl.BlockSpec((R, H, H), lambda i: (0, 0, 0)),
            pl.BlockSpec((H, H), lambda i: (0, 0)),
            pl.BlockSpec((1, H), lambda i: (0, 0)),
        ],
        out_specs=[
            pl.BlockSpec((BM, H), lambda i: (i, 0)),
            pl.BlockSpec((BM, R, H), lambda i: (i, 0, 0)),
        ],
        out_shape=[
            jax.ShapeDtypeStruct((N, H), jnp.float32),
            jax.ShapeDtypeStruct((N, R, H), jnp.float32),
        ],
    )(rt, part, w, root, b)


def _tc_final(rt, part):
    def body(rt_ref, p_ref, o_ref):
        o_ref[...] = jnp.maximum(rt_ref[...] + p_ref[0] + p_ref[1], 0.0)

    return pl.pallas_call(
        body,
        grid=(N // BM,),
        in_specs=[
            pl.BlockSpec((BM, H), lambda i: (i, 0)),
            pl.BlockSpec((NC, BM, H), lambda i: (0, i, 0)),
        ],
        out_specs=pl.BlockSpec((BM, H), lambda i: (i, 0)),
        out_shape=jax.ShapeDtypeStruct((N, H), jnp.float32),
    )(rt, part)


def kernel(x, edge_index, edge_type, w1, root1, b1, w2, root2, b2):
    src = edge_index[0]
    dst = edge_index[1]
    dst2 = dst.reshape(E // SB, SB)
    zflat = jnp.zeros((8000,), jnp.float32)
    zrows = jnp.zeros((ZROWS, H), jnp.float32)

    srckey, dstkey, hist = _sc_prep(src, dst, edge_type, zflat)
    scale = _sc_scale(dstkey, hist)

    rt1, y1 = _tc_lin1(x, w1, root1, b1.reshape(1, H))
    p1 = _sc_edge(y1.reshape(KN, H), srckey, dst2, scale, zrows)
    rt2, y2 = _tc_comb(rt1, p1, w2, root2, b2.reshape(1, H))
    p2 = _sc_edge(y2.reshape(KN, H), srckey, dst2, scale, zrows)
    return _tc_final(rt2, p2)


# speedup vs baseline: 1.0315x; 1.0315x over previous
import functools

import jax
import jax.numpy as jnp
from jax import lax
from jax.experimental import pallas as pl
from jax.experimental.pallas import tpu as pltpu
from jax.experimental.pallas import tpu_sc as plsc

N = 10000
E = 320000
H = 64
R = 8
KN = N * R

NC = 2
NS = 16
NW = NC * NS
L = 16

SB = 80
C = 400
SUBS = C // SB
EPW = E // NW
NCHUNK = EPW // C
CP = 2000
ZROWS = 125
RPS = N // NS


def _sc_mesh():
    return plsc.VectorSubcoreMesh(
        core_axis_name="c", subcore_axis_name="s",
        num_cores=NC, num_subcores=NS)


_SC_PARAMS = pltpu.CompilerParams(
    needs_layout_passes=False, use_tc_tiling_on_sc=False)


@functools.partial(
    pl.kernel,
    out_type=(
        jax.ShapeDtypeStruct((E,), jnp.int32),
        jax.ShapeDtypeStruct((E,), jnp.int32),
        jax.ShapeDtypeStruct((NW, KN), jnp.float32),
    ),
    mesh=_sc_mesh(),
    scratch_types=[
        pltpu.VMEM((KN,), jnp.float32),
        pltpu.VMEM((CP,), jnp.int32),
        pltpu.VMEM((CP,), jnp.int32),
        pltpu.VMEM((CP,), jnp.int32),
        pltpu.VMEM((CP,), jnp.int32),
        pltpu.VMEM((CP,), jnp.int32),
    ],
    compiler_params=_SC_PARAMS,
)
def _sc_prep(src, dst, et, zflat, srckey, dstkey, hist,
             hist_v, src_v, dst_v, typ_v, sk_v, dk_v):
    cc = lax.axis_index("c")
    ss = lax.axis_index("s")
    wid = ss * NC + cc
    base = wid * EPW
    for i in range(KN // 8000):
        pltpu.sync_copy(zflat, hist_v.at[pl.ds(i * 8000, 8000)])
    ones = jnp.ones((L,), jnp.float32)

    def chunk(i, carry):
        off = base + i * CP
        pltpu.sync_copy(src.at[pl.ds(off, CP)], src_v)
        pltpu.sync_copy(dst.at[pl.ds(off, CP)], dst_v)
        pltpu.sync_copy(et.at[pl.ds(off, CP)], typ_v)
        for k in range(CP // L):
            sl = pl.ds(k * L, L)
            t = typ_v[sl]
            sk_v[sl] = src_v[sl] * R + t
            dk = dst_v[sl] * R + t
            dk_v[sl] = dk
            plsc.addupdate_scatter(hist_v, [dk], ones)
        pltpu.sync_copy(sk_v, srckey.at[pl.ds(off, CP)])
        pltpu.sync_copy(dk_v, dstkey.at[pl.ds(off, CP)])
        return carry

    lax.fori_loop(0, EPW // CP, chunk, 0)
    pltpu.sync_copy(hist_v, hist.at[wid])


@functools.partial(
    pl.kernel,
    out_type=jax.ShapeDtypeStruct((E,), jnp.float32),
    mesh=_sc_mesh(),
    scratch_types=[
        pltpu.VMEM((KN,), jnp.float32),
        pltpu.VMEM((CP,), jnp.int32),
        pltpu.VMEM((CP,), jnp.float32),
    ],
    compiler_params=_SC_PARAMS,
)
def _sc_scale(dstkey, recip, scale, recip_v, dk_v, s_v):
    cc = lax.axis_index("c")
    ss = lax.axis_index("s")
    wid = ss * NC + cc
    base = wid * EPW
    pltpu.sync_copy(recip, recip_v)

    def chunk(i, carry):
        off = base + i * CP
        pltpu.sync_copy(dstkey.at[pl.ds(off, CP)], dk_v)
        for k in range(CP // L):
            sl = pl.ds(k * L, L)
            s_v[sl] = plsc.load_gather(recip_v, [dk_v[sl]])
        pltpu.sync_copy(s_v, scale.at[pl.ds(off, CP)])
        return carry

    lax.fori_loop(0, EPW // CP, chunk, 0)


@functools.partial(
    pl.kernel,
    out_type=jax.ShapeDtypeStruct((NC, N, H), jnp.float32),
    mesh=_sc_mesh(),
    scratch_types=[
        pltpu.VMEM((C, H), jnp.float32),
        pltpu.VMEM((C, H), jnp.float32),
        pltpu.VMEM((C, H), jnp.float32),
        pltpu.VMEM((C,), jnp.int32),
        pltpu.VMEM((C,), jnp.int32),
        pltpu.VMEM((C,), jnp.int32),
        pltpu.VMEM((SUBS, SB), jnp.int32),
        pltpu.VMEM((SUBS, SB), jnp.int32),
        pltpu.VMEM((SUBS, SB), jnp.int32),
        pltpu.VMEM((C,), jnp.float32),
        pltpu.VMEM((C,), jnp.float32),
        pltpu.VMEM((C,), jnp.float32),
        pltpu.VMEM_SHARED((N, H), jnp.float32),
        pltpu.SemaphoreType.DMA,
        pltpu.SemaphoreType.DMA,
        pltpu.SemaphoreType.DMA,
        pltpu.SemaphoreType.DMA,
        pltpu.SemaphoreType.DMA,
        pltpu.SemaphoreType.DMA,
    ],
    compiler_params=_SC_PARAMS,
)
def _sc_edge(y, srckey, dst2, scale, zrows, out,
             rows_a, rows_b, rows_z, sk_a, sk_b, sk_z, dk_a, dk_b, dk_z,
             s_a, s_b, s_z, agg_sh, gsem_a, gsem_b, gsem_z,
             ssem_a, ssem_b, ssem_z):
    cc = lax.axis_index("c")
    ss = lax.axis_index("s")
    wid = ss * NC + cc
    base = wid * EPW
    for j in range(RPS // ZROWS):
        pltpu.sync_copy(zrows, agg_sh.at[pl.ds(ss * RPS + j * ZROWS, ZROWS)])
    plsc.subcore_barrier()

    def loadmeta(n, sk_v, dk_v, s_v):
        off = base + n * C
        row = (base + n * C) // SB
        pltpu.sync_copy(srckey.at[pl.ds(off, C)], sk_v)
        pltpu.sync_copy(dst2.at[pl.ds(row, SUBS)], dk_v)
        pltpu.sync_copy(scale.at[pl.ds(off, C)], s_v)

    def gather_descs(sk_v, rows_v, sem):
        return [
            pltpu.make_async_copy(
                y.at[sk_v.at[pl.ds(j * SB, SB)]],
                rows_v.at[pl.ds(j * SB, SB)], sem)
            for j in range(SUBS)
        ]

    def scatter_descs(rows_v, dk_v, sem):
        return [
            pltpu.make_async_copy(
                rows_v.at[pl.ds(j * SB, SB)],
                agg_sh.at[dk_v.at[j]], sem)
            for j in range(SUBS)
        ]

    def issue(descs, add=False):
        for d in descs:
            d.start(add=add)

    def drain(descs):
        for d in descs:
            d.wait()

    def mul_rows(rows_v, s_v):
        def body(e):
            eidx = jnp.full((L,), e, jnp.int32)
            sv = plsc.load_gather(s_v, [eidx])
            for j in range(H // L):
                sl = pl.ds(j * L, L)
                rows_v[e, sl] = rows_v[e, sl] * sv
        plsc.parallel_loop(0, C, 1, unroll=16)(body)

    BUFS = ((rows_a, sk_a, dk_a, s_a, gsem_a, ssem_a),
            (rows_b, sk_b, dk_b, s_b, gsem_b, ssem_b),
            (rows_z, sk_z, dk_z, s_z, gsem_z, ssem_z))

    def refill(n, buf):
        rows_v, sk_v, dk_v, s_v, gsem, _ = buf
        loadmeta(n, sk_v, dk_v, s_v)
        issue(gather_descs(sk_v, rows_v, gsem))

    def process(buf):
        rows_v, sk_v, dk_v, s_v, gsem, ssem = buf
        drain(gather_descs(sk_v, rows_v, gsem))
        mul_rows(rows_v, s_v)
        issue(scatter_descs(rows_v, dk_v, ssem), add=True)

    def wait_scat(buf):
        rows_v, _, dk_v, _, _, ssem = buf
        drain(scatter_descs(rows_v, dk_v, ssem))

    refill(0, BUFS[0])
    refill(1, BUFS[1])

    def trip(i, carry):
        a, b, z = BUFS
        process(a)

        @pl.when(i > 0)
        def _():
            wait_scat(z)
        refill(3 * i + 2, z)

        process(b)

        @pl.when(3 * i + 3 < NCHUNK)
        def _():
            wait_scat(a)
            refill(3 * i + 3, a)

        process(z)

        @pl.when(3 * i + 4 < NCHUNK)
        def _():
            wait_scat(b)
            refill(3 * i + 4, b)
        return carry

    lax.fori_loop(0, NCHUNK // 3, trip, 0)
    process(BUFS[0])
    wait_scat(BUFS[0])
    wait_scat(BUFS[1])
    wait_scat(BUFS[2])
    plsc.subcore_barrier()

    @pl.when(ss == 0)
    def _():
        pltpu.sync_copy(agg_sh, out.at[cc])


BM = 1000


def _recip_body(h_ref, r_ref):
    s = jnp.sum(h_ref[...], axis=0, keepdims=True)
    r_ref[...] = 1.0 / jnp.maximum(s, 1.0)


def _tc_recip(hist):
    out = pl.pallas_call(
        _recip_body,
        grid=(KN // 16000,),
        in_specs=[pl.BlockSpec((NW, 16000), lambda i: (0, i))],
        out_specs=pl.BlockSpec((1, 16000), lambda i: (0, i)),
        out_shape=jax.ShapeDtypeStruct((1, KN), jnp.float32),
    )(hist)
    return out.reshape(KN)


def _tc_lin1(x, w, root, b):
    def body(x_ref, w_ref, root_ref, b_ref, rt_ref, y_ref):
        feat = jnp.concatenate([x_ref[0], x_ref[1]], axis=1)
        rt_ref[...] = jnp.dot(
            feat, root_ref[...], preferred_element_type=jnp.float32
        ) + b_ref[...]
        for r in range(R):
            y_ref[:, r, :] = jnp.dot(
                feat, w_ref[r], preferred_element_type=jnp.float32)

    return pl.pallas_call(
        body,
        grid=(N // BM,),
        in_specs=[
            pl.BlockSpec((2, BM, H), lambda i: (0, i, 0)),
            pl.BlockSpec((R, 2 * H, H), lambda i: (0, 0, 0)),
            pl.BlockSpec((2 * H, H), lambda i: (0, 0)),
            pl.BlockSpec((1, H), lambda i: (0, 0)),
        ],
        out_specs=[
            pl.BlockSpec((BM, H), lambda i: (i, 0)),
            pl.BlockSpec((BM, R, H), lambda i: (i, 0, 0)),
        ],
        out_shape=[
            jax.ShapeDtypeStruct((N, H), jnp.float32),
            jax.ShapeDtypeStruct((N, R, H), jnp.float32),
        ],
    )(x, w, root, b)


def _tc_comb(rt, part, w, root, b):
    def body(rt_ref, p_ref, w_ref, root_ref, b_ref, rt2_ref, y2_ref):
        h = jnp.maximum(rt_ref[...] + p_ref[0] + p_ref[1], 0.0)
        rt2_ref[...] = jnp.dot(
            h, root_ref[...], preferred_element_type=jnp.float32
        ) + b_ref[...]
        for r in range(R):
            y2_ref[:, r, :] = jnp.dot(
                h, w_ref[r], preferred_element_type=jnp.float32)

    return pl.pallas_call(
        body,
        grid=(N // BM,),
        in_specs=[
            pl.BlockSpec((BM, H), lambda i: (i, 0)),
            pl.BlockSpec((NC, BM, H), lambda i: (0, i, 0)),
            pl.BlockSpec((R, H, H), lambda i: (0, 0, 0)),
            pl.BlockSpec((H, H), lambda i: (0, 0)),
            pl.BlockSpec((1, H), lambda i: (0, 0)),
        ],
        out_specs=[
            pl.BlockSpec((BM, H), lambda i: (i, 0)),
            pl.BlockSpec((BM, R, H), lambda i: (i, 0, 0)),
        ],
        out_shape=[
            jax.ShapeDtypeStruct((N, H), jnp.float32),
            jax.ShapeDtypeStruct((N, R, H), jnp.float32),
        ],
    )(rt, part, w, root, b)


def _tc_final(rt, part):
    def body(rt_ref, p_ref, o_ref):
        o_ref[...] = jnp.maximum(rt_ref[...] + p_ref[0] + p_ref[1], 0.0)

    return pl.pallas_call(
        body,
        grid=(N // BM,),
        in_specs=[
            pl.BlockSpec((BM, H), lambda i: (i, 0)),
            pl.BlockSpec((NC, BM, H), lambda i: (0, i, 0)),
        ],
        out_specs=pl.BlockSpec((BM, H), lambda i: (i, 0)),
        out_shape=jax.ShapeDtypeStruct((N, H), jnp.float32),
    )(rt, part)


def kernel(x, edge_index, edge_type, w1, root1, b1, w2, root2, b2):
    src = edge_index[0]
    dst = edge_index[1]
    dst2 = dst.reshape(E // SB, SB)
    zflat = jnp.zeros((8000,), jnp.float32)
    zrows = jnp.zeros((ZROWS, H), jnp.float32)

    srckey, dstkey, hist = _sc_prep(src, dst, edge_type, zflat)
    recip = _tc_recip(hist)
    scale = _sc_scale(dstkey, recip)

    rt1, y1 = _tc_lin1(x, w1, root1, b1.reshape(1, H))
    p1 = _sc_edge(y1.reshape(KN, H), srckey, dst2, scale, zrows)
    rt2, y2 = _tc_comb(rt1, p1, w2, root2, b2.reshape(1, H))
    p2 = _sc_edge(y2.reshape(KN, H), srckey, dst2, scale, zrows)
    return _tc_final(rt2, p2)


# speedup vs baseline: 1.0396x; 1.0079x over previous
import functools

import jax
import jax.numpy as jnp
from jax import lax
from jax.experimental import pallas as pl
from jax.experimental.pallas import tpu as pltpu
from jax.experimental.pallas import tpu_sc as plsc

N = 10000
E = 320000
H = 64
R = 8
KN = N * R

NC = 2
NS = 16
NW = NC * NS
L = 16

SB = 80
C = 400
SUBS = C // SB
EPW = E // NW
NCHUNK = EPW // C
CP = 2000
ZROWS = 125
RPS = N // NS


def _sc_mesh():
    return plsc.VectorSubcoreMesh(
        core_axis_name="c", subcore_axis_name="s",
        num_cores=NC, num_subcores=NS)


_SC_PARAMS = pltpu.CompilerParams(
    needs_layout_passes=False, use_tc_tiling_on_sc=False)


@functools.partial(
    pl.kernel,
    out_type=(
        jax.ShapeDtypeStruct((E,), jnp.int32),
        jax.ShapeDtypeStruct((E,), jnp.int32),
        jax.ShapeDtypeStruct((NW, KN), jnp.float32),
    ),
    mesh=_sc_mesh(),
    scratch_types=[
        pltpu.VMEM((KN,), jnp.float32),
        pltpu.VMEM((CP,), jnp.int32),
        pltpu.VMEM((CP,), jnp.int32),
        pltpu.VMEM((CP,), jnp.int32),
        pltpu.VMEM((CP,), jnp.int32),
        pltpu.VMEM((CP,), jnp.int32),
    ],
    compiler_params=_SC_PARAMS,
)
def _sc_prep(src, dst, et, zflat, srckey, dstkey, hist,
             hist_v, src_v, dst_v, typ_v, sk_v, dk_v):
    cc = lax.axis_index("c")
    ss = lax.axis_index("s")
    wid = ss * NC + cc
    base = wid * EPW
    for i in range(KN // 8000):
        pltpu.sync_copy(zflat, hist_v.at[pl.ds(i * 8000, 8000)])
    ones = jnp.ones((L,), jnp.float32)

    def chunk(i, carry):
        off = base + i * CP
        pltpu.sync_copy(src.at[pl.ds(off, CP)], src_v)
        pltpu.sync_copy(dst.at[pl.ds(off, CP)], dst_v)
        pltpu.sync_copy(et.at[pl.ds(off, CP)], typ_v)
        for k in range(CP // L):
            sl = pl.ds(k * L, L)
            t = typ_v[sl]
            sk_v[sl] = src_v[sl] * R + t
            dk = dst_v[sl] * R + t
            dk_v[sl] = dk
            plsc.addupdate_scatter(hist_v, [dk], ones)
        pltpu.sync_copy(sk_v, srckey.at[pl.ds(off, CP)])
        pltpu.sync_copy(dk_v, dstkey.at[pl.ds(off, CP)])
        return carry

    lax.fori_loop(0, EPW // CP, chunk, 0)
    pltpu.sync_copy(hist_v, hist.at[wid])


@functools.partial(
    pl.kernel,
    out_type=jax.ShapeDtypeStruct((E,), jnp.float32),
    mesh=_sc_mesh(),
    scratch_types=[
        pltpu.VMEM((KN,), jnp.float32),
        pltpu.VMEM((CP,), jnp.int32),
        pltpu.VMEM((CP,), jnp.float32),
    ],
    compiler_params=_SC_PARAMS,
)
def _sc_scale(dstkey, recip, scale, recip_v, dk_v, s_v):
    cc = lax.axis_index("c")
    ss = lax.axis_index("s")
    wid = ss * NC + cc
    base = wid * EPW
    pltpu.sync_copy(recip, recip_v)

    def chunk(i, carry):
        off = base + i * CP
        pltpu.sync_copy(dstkey.at[pl.ds(off, CP)], dk_v)
        for k in range(CP // L):
            sl = pl.ds(k * L, L)
            s_v[sl] = plsc.load_gather(recip_v, [dk_v[sl]])
        pltpu.sync_copy(s_v, scale.at[pl.ds(off, CP)])
        return carry

    lax.fori_loop(0, EPW // CP, chunk, 0)


@functools.partial(
    pl.kernel,
    out_type=jax.ShapeDtypeStruct((NC, N, H), jnp.float32),
    mesh=_sc_mesh(),
    scratch_types=[
        pltpu.VMEM((C, H), jnp.float32),
        pltpu.VMEM((C, H), jnp.float32),
        pltpu.VMEM((C, H), jnp.float32),
        pltpu.VMEM((C,), jnp.int32),
        pltpu.VMEM((C,), jnp.int32),
        pltpu.VMEM((C,), jnp.int32),
        pltpu.VMEM((SUBS, SB), jnp.int32),
        pltpu.VMEM((SUBS, SB), jnp.int32),
        pltpu.VMEM((SUBS, SB), jnp.int32),
        pltpu.VMEM((C,), jnp.float32),
        pltpu.VMEM((C,), jnp.float32),
        pltpu.VMEM((C,), jnp.float32),
        pltpu.VMEM_SHARED((N, H), jnp.float32),
        pltpu.SemaphoreType.DMA,
        pltpu.SemaphoreType.DMA,
        pltpu.SemaphoreType.DMA,
        pltpu.SemaphoreType.DMA,
        pltpu.SemaphoreType.DMA,
        pltpu.SemaphoreType.DMA,
    ],
    compiler_params=_SC_PARAMS,
)
def _sc_edge(y, srckey, dst2, scale, zrows, out,
             rows_a, rows_b, rows_z, sk_a, sk_b, sk_z, dk_a, dk_b, dk_z,
             s_a, s_b, s_z, agg_sh, gsem_a, gsem_b, gsem_z,
             ssem_a, ssem_b, ssem_z):
    cc = lax.axis_index("c")
    ss = lax.axis_index("s")
    wid = ss * NC + cc
    base = wid * EPW
    for j in range(RPS // ZROWS):
        pltpu.sync_copy(zrows, agg_sh.at[pl.ds(ss * RPS + j * ZROWS, ZROWS)])
    plsc.subcore_barrier()

    def loadmeta(n, sk_v, dk_v, s_v):
        off = base + n * C
        row = (base + n * C) // SB
        pltpu.sync_copy(srckey.at[pl.ds(off, C)], sk_v)
        pltpu.sync_copy(dst2.at[pl.ds(row, SUBS)], dk_v)
        pltpu.sync_copy(scale.at[pl.ds(off, C)], s_v)

    def gather_descs(sk_v, rows_v, sem):
        return [
            pltpu.make_async_copy(
                y.at[sk_v.at[pl.ds(j * SB, SB)]],
                rows_v.at[pl.ds(j * SB, SB)], sem)
            for j in range(SUBS)
        ]

    def scatter_descs(rows_v, dk_v, sem):
        return [
            pltpu.make_async_copy(
                rows_v.at[pl.ds(j * SB, SB)],
                agg_sh.at[dk_v.at[j]], sem)
            for j in range(SUBS)
        ]

    def issue(descs, add=False):
        for d in descs:
            d.start(add=add)

    def drain(descs):
        for d in descs:
            d.wait()

    def mul_rows(rows_v, s_v):
        def body(e):
            eidx = jnp.full((L,), e, jnp.int32)
            sv = plsc.load_gather(s_v, [eidx])
            for j in range(H // L):
                sl = pl.ds(j * L, L)
                rows_v[e, sl] = rows_v[e, sl] * sv
        plsc.parallel_loop(0, C, 1, unroll=16)(body)

    BUFS = ((rows_a, sk_a, dk_a, s_a, gsem_a, ssem_a),
            (rows_b, sk_b, dk_b, s_b, gsem_b, ssem_b),
            (rows_z, sk_z, dk_z, s_z, gsem_z, ssem_z))

    def refill(n, buf):
        rows_v, sk_v, dk_v, s_v, gsem, _ = buf
        loadmeta(n, sk_v, dk_v, s_v)
        issue(gather_descs(sk_v, rows_v, gsem))

    def process(buf):
        rows_v, sk_v, dk_v, s_v, gsem, ssem = buf
        drain(gather_descs(sk_v, rows_v, gsem))
        mul_rows(rows_v, s_v)
        issue(scatter_descs(rows_v, dk_v, ssem), add=True)

    def wait_scat(buf):
        rows_v, _, dk_v, _, _, ssem = buf
        drain(scatter_descs(rows_v, dk_v, ssem))

    refill(0, BUFS[0])
    refill(1, BUFS[1])

    def trip(i, carry):
        a, b, z = BUFS
        process(a)

        @pl.when(i > 0)
        def _():
            wait_scat(z)
        refill(3 * i + 2, z)

        process(b)

        @pl.when(3 * i + 3 < NCHUNK)
        def _():
            wait_scat(a)
            refill(3 * i + 3, a)

        process(z)

        @pl.when(3 * i + 4 < NCHUNK)
        def _():
            wait_scat(b)
            refill(3 * i + 4, b)
        return carry

    lax.fori_loop(0, NCHUNK // 3, trip, 0)
    process(BUFS[0])
    wait_scat(BUFS[0])
    wait_scat(BUFS[1])
    wait_scat(BUFS[2])
    plsc.subcore_barrier()

    @pl.when(ss == 0)
    def _():
        pltpu.sync_copy(agg_sh, out.at[cc])


BM = 1000


def _tc_lin1(x, w, root, b, hist):
    def body(x_ref, w_ref, root_ref, b_ref, h_ref, rt_ref, y_ref, rec_ref):
        feat = jnp.concatenate([x_ref[0], x_ref[1]], axis=1)
        rt_ref[...] = jnp.dot(
            feat, root_ref[...], preferred_element_type=jnp.float32
        ) + b_ref[...]
        for r in range(R):
            y_ref[:, r, :] = jnp.dot(
                feat, w_ref[r], preferred_element_type=jnp.float32)
        s = jnp.sum(h_ref[...], axis=0, keepdims=True)
        rec_ref[...] = 1.0 / jnp.maximum(s, 1.0)

    return pl.pallas_call(
        body,
        grid=(N // BM,),
        in_specs=[
            pl.BlockSpec((2, BM, H), lambda i: (0, i, 0)),
            pl.BlockSpec((R, 2 * H, H), lambda i: (0, 0, 0)),
            pl.BlockSpec((2 * H, H), lambda i: (0, 0)),
            pl.BlockSpec((1, H), lambda i: (0, 0)),
            pl.BlockSpec((NW, 1, 1, 16000), lambda i: (0, i % 5, 0, 0)),
        ],
        out_specs=[
            pl.BlockSpec((BM, H), lambda i: (i, 0)),
            pl.BlockSpec((BM, R, H), lambda i: (i, 0, 0)),
            pl.BlockSpec((1, 1, 1, 16000), lambda i: (0, i % 5, 0, 0)),
        ],
        out_shape=[
            jax.ShapeDtypeStruct((N, H), jnp.float32),
            jax.ShapeDtypeStruct((N, R, H), jnp.float32),
            jax.ShapeDtypeStruct((1, 5, 1, 16000), jnp.float32),
        ],
    )(x, w, root, b, hist)


def _tc_comb(rt, part, w, root, b):
    def body(rt_ref, p_ref, w_ref, root_ref, b_ref, rt2_ref, y2_ref):
        h = jnp.maximum(rt_ref[...] + p_ref[0] + p_ref[1], 0.0)
        rt2_ref[...] = jnp.dot(
            h, root_ref[...], preferred_element_type=jnp.float32
        ) + b_ref[...]
        for r in range(R):
            y2_ref[:, r, :] = jnp.dot(
                h, w_ref[r], preferred_element_type=jnp.float32)

    return pl.pallas_call(
        body,
        grid=(N // BM,),
        in_specs=[
            pl.BlockSpec((BM, H), lambda i: (i, 0)),
            pl.BlockSpec((NC, BM, H), lambda i: (0, i, 0)),
            pl.BlockSpec((R, H, H), lambda i: (0, 0, 0)),
            pl.BlockSpec((H, H), lambda i: (0, 0)),
            pl.BlockSpec((1, H), lambda i: (0, 0)),
        ],
        out_specs=[
            pl.BlockSpec((BM, H), lambda i: (i, 0)),
            pl.BlockSpec((BM, R, H), lambda i: (i, 0, 0)),
        ],
        out_shape=[
            jax.ShapeDtypeStruct((N, H), jnp.float32),
            jax.ShapeDtypeStruct((N, R, H), jnp.float32),
        ],
    )(rt, part, w, root, b)


def _tc_final(rt, part):
    def body(rt_ref, p_ref, o_ref):
        o_ref[...] = jnp.maximum(rt_ref[...] + p_ref[0] + p_ref[1], 0.0)

    return pl.pallas_call(
        body,
        grid=(N // BM,),
        in_specs=[
            pl.BlockSpec((BM, H), lambda i: (i, 0)),
            pl.BlockSpec((NC, BM, H), lambda i: (0, i, 0)),
        ],
        out_specs=pl.BlockSpec((BM, H), lambda i: (i, 0)),
        out_shape=jax.ShapeDtypeStruct((N, H), jnp.float32),
    )(rt, part)


def kernel(x, edge_index, edge_type, w1, root1, b1, w2, root2, b2):
    src = edge_index[0]
    dst = edge_index[1]
    dst2 = dst.reshape(E // SB, SB)
    zflat = jnp.zeros((8000,), jnp.float32)
    zrows = jnp.zeros((ZROWS, H), jnp.float32)

    srckey, dstkey, hist = _sc_prep(src, dst, edge_type, zflat)

    rt1, y1, recip = _tc_lin1(x, w1, root1, b1.reshape(1, H),
                              hist.reshape(NW, 5, 1, 16000))
    scale = _sc_scale(dstkey, recip.reshape(KN))
    p1 = _sc_edge(y1.reshape(KN, H), srckey, dst2, scale, zrows)
    rt2, y2 = _tc_comb(rt1, p1, w2, root2, b2.reshape(1, H))
    p2 = _sc_edge(y2.reshape(KN, H), srckey, dst2, scale, zrows)
    return _tc_final(rt2, p2)
